# Initial kernel scaffold; baseline (speedup 1.0000x reference)
import jax
import jax.numpy as jnp
from jax.experimental import pallas as pl


def kernel(input_ids, embed_table, W, b):
    raise NotImplementedError("write your pallas kernel here")


# speedup vs baseline: 2.5905x; 2.5905x over previous
import functools

import jax
import jax.numpy as jnp
from jax import lax
from jax.experimental import pallas as pl
from jax.experimental.pallas import tpu as pltpu
from jax.experimental.pallas import tpu_sc as plsc

VOCAB = 64
EMBED_DIM = 16

NC = 2
NS = 16
NW = NC * NS

IDX_W = 128
G = 4
CHUNK = G * IDX_W
LANES = 16


def _ptable_body(e_ref, w_ref, b_ref, o_ref):
    t = lax.dot_general(
        e_ref[...], w_ref[...], (((1,), (1,)), ((), ())),
        preferred_element_type=jnp.float32,
    ) + b_ref[...]
    o_ref[:, :, 0, :] = jnp.broadcast_to(t[:, None, :], (VOCAB, VOCAB, VOCAB))
    o_ref[:, :, 1, :] = jnp.broadcast_to(t[None, :, :], (VOCAB, VOCAB, VOCAB))


def _pair_table(embed_table, W, b2d):
    out = pl.pallas_call(
        _ptable_body,
        out_shape=jax.ShapeDtypeStruct((VOCAB, VOCAB, 2, VOCAB), jnp.float32),
    )(embed_table, W, b2d)
    return out.reshape(VOCAB * VOCAB, 2 * VOCAB)


def _make_sc_gather(n_chunks):
    mesh = plsc.VectorSubcoreMesh(core_axis_name="c", subcore_axis_name="s")

    @functools.partial(
        pl.kernel,
        mesh=mesh,
        out_type=jax.ShapeDtypeStruct((NW, n_chunks, G, IDX_W, 2 * VOCAB),
                                      jnp.float32),
        scratch_types=[
            pltpu.VMEM((G, IDX_W), jnp.int32),
            pltpu.VMEM((G, IDX_W), jnp.int32),
            pltpu.VMEM((G, IDX_W), jnp.int32),
            pltpu.VMEM((G, IDX_W, 2 * VOCAB), jnp.float32),
            pltpu.SemaphoreType.DMA,
        ],
    )
    def sc_gather(ptable_hbm, ev_hbm, od_hbm, out_hbm, ev_v, od_v, idx_v,
                  rows_v, sem):
        w = lax.axis_index("s") * NC + lax.axis_index("c")

        def chunk_body(c, carry):
            pltpu.sync_copy(ev_hbm.at[w, c], ev_v)
            pltpu.sync_copy(od_hbm.at[w, c], od_v)
            for j in range(G):
                for k in range(IDX_W // LANES):
                    sl = pl.ds(k * LANES, LANES)
                    idx_v[j, sl] = ev_v[j, sl] * VOCAB + od_v[j, sl]
            cps = [
                pltpu.async_copy(ptable_hbm.at[idx_v.at[j]], rows_v.at[j], sem)
                for j in range(G)
            ]
            for cp in cps:
                cp.wait()
            pltpu.sync_copy(rows_v, out_hbm.at[w, c])
            return carry

        lax.fori_loop(0, n_chunks, chunk_body, 0)

    return sc_gather


def kernel(input_ids, embed_table, W, b):
    B, T = input_ids.shape
    n_pairs = B * T // 2
    assert n_pairs % (NW * CHUNK) == 0
    n_chunks = n_pairs // (NW * CHUNK)

    ptable = _pair_table(embed_table, W, b.reshape(1, VOCAB))
    ids2 = input_ids.reshape(NW, n_chunks, G, IDX_W, 2).astype(jnp.int32)
    out = _make_sc_gather(n_chunks)(ptable, ids2[..., 0], ids2[..., 1])
    return out.reshape(B, T, VOCAB)


# speedup vs baseline: 2.6764x; 1.0332x over previous
import functools

import jax
import jax.numpy as jnp
from jax import lax
from jax.experimental import pallas as pl
from jax.experimental.pallas import tpu as pltpu
from jax.experimental.pallas import tpu_sc as plsc

VOCAB = 64
EMBED_DIM = 16

NC = 2
NS = 16
NW = NC * NS

IDX_W = 128
G = 2
CHUNK = G * IDX_W
NSLOT = 2
LANES = 16


def _ptable_body(e_ref, w_ref, b_ref, o_ref):
    t = lax.dot_general(
        e_ref[...], w_ref[...], (((1,), (1,)), ((), ())),
        preferred_element_type=jnp.float32,
    ) + b_ref[...]
    o_ref[:, :, 0, :] = jnp.broadcast_to(t[:, None, :], (VOCAB, VOCAB, VOCAB))
    o_ref[:, :, 1, :] = jnp.broadcast_to(t[None, :, :], (VOCAB, VOCAB, VOCAB))


def _pair_table(embed_table, W, b2d):
    out = pl.pallas_call(
        _ptable_body,
        out_shape=jax.ShapeDtypeStruct((VOCAB, VOCAB, 2, VOCAB), jnp.float32),
    )(embed_table, W, b2d)
    return out.reshape(VOCAB * VOCAB, 2 * VOCAB)


def _make_sc_gather(n_chunks):
    mesh = plsc.VectorSubcoreMesh(core_axis_name="c", subcore_axis_name="s")
    assert n_chunks % NSLOT == 0

    @functools.partial(
        pl.kernel,
        mesh=mesh,
        out_type=jax.ShapeDtypeStruct((NW, n_chunks, G, IDX_W, 2 * VOCAB),
                                      jnp.float32),
        scratch_types=[
            pltpu.VMEM((NSLOT, G, IDX_W), jnp.int32),
            pltpu.VMEM((NSLOT, G, IDX_W), jnp.int32),
            pltpu.VMEM((NSLOT, G, IDX_W), jnp.int32),
            pltpu.VMEM((NSLOT, G, IDX_W, 2 * VOCAB), jnp.float32),
            pltpu.SemaphoreType.DMA((NSLOT,)),
            pltpu.SemaphoreType.DMA((NSLOT,)),
            pltpu.SemaphoreType.DMA((NSLOT,)),
        ],
    )
    def sc_gather(ptable_hbm, ev_hbm, od_hbm, out_hbm, ev_v, od_v, idx_v,
                  rows_v, sem_ids, sem_g, sem_out):
        w = lax.axis_index("s") * NC + lax.axis_index("c")

        def fire_ids(c, s):
            pltpu.async_copy(ev_hbm.at[w, c], ev_v.at[s], sem_ids.at[s])
            pltpu.async_copy(od_hbm.at[w, c], od_v.at[s], sem_ids.at[s])

        def wait_ids(c, s):
            pltpu.make_async_copy(ev_hbm.at[w, c], ev_v.at[s],
                                  sem_ids.at[s]).wait()
            pltpu.make_async_copy(od_hbm.at[w, c], od_v.at[s],
                                  sem_ids.at[s]).wait()

        def wait_out(c, s):
            pltpu.make_async_copy(rows_v.at[s], out_hbm.at[w, c],
                                  sem_out.at[s]).wait()

        for s in range(NSLOT):
            fire_ids(s, s)

        def body(g, carry):
            for s in range(NSLOT):
                c = g * NSLOT + s

                @pl.when(g > 0)
                def _():
                    wait_out(c - NSLOT, s)

                wait_ids(c, s)
                for j in range(G):
                    for k in range(IDX_W // LANES):
                        sl = pl.ds(k * LANES, LANES)
                        idx_v[s, j, sl] = ev_v[s, j, sl] * VOCAB + od_v[s, j, sl]

                @pl.when(g < n_chunks // NSLOT - 1)
                def _():
                    fire_ids(c + NSLOT, s)

                cps = [
                    pltpu.async_copy(ptable_hbm.at[idx_v.at[s, j]],
                                     rows_v.at[s, j], sem_g.at[s])
                    for j in range(G)
                ]
                for cp in cps:
                    cp.wait()
                pltpu.async_copy(rows_v.at[s], out_hbm.at[w, c], sem_out.at[s])
            return carry

        lax.fori_loop(0, n_chunks // NSLOT, body, 0)
        for s in range(NSLOT):
            wait_out(n_chunks - NSLOT + s, s)

    return sc_gather


def kernel(input_ids, embed_table, W, b):
    B, T = input_ids.shape
    n_pairs = B * T // 2
    assert n_pairs % (NW * CHUNK) == 0
    n_chunks = n_pairs // (NW * CHUNK)

    ptable = _pair_table(embed_table, W, b.reshape(1, VOCAB))
    ids2 = input_ids.reshape(NW, n_chunks, G, IDX_W, 2).astype(jnp.int32)
    out = _make_sc_gather(n_chunks)(ptable, ids2[..., 0], ids2[..., 1])
    return out.reshape(B, T, VOCAB)
